# baseline (speedup 1.0000x reference)
import functools

import jax
import jax.numpy as jnp
from jax import lax
from jax.experimental import pallas as pl
from jax.experimental.pallas import tpu as pltpu
from jax.experimental.pallas import tpu_sc as plsc

L_CTX = 2048
D = 128
B = 4
N = B * L_CTX
NC = 2
NS = 16
NW = NC * NS
P_W = L_CTX // NW
W_CH = 32
N_WCH = P_W // W_CH
LANES = 16

_mesh = plsc.VectorSubcoreMesh(core_axis_name="c", subcore_axis_name="s")


@functools.partial(
    pl.kernel,
    out_type=jax.ShapeDtypeStruct((N, D), jnp.float32),
    mesh=_mesh,
    scratch_types=[
        pltpu.VMEM((B, P_W), jnp.int32),
        pltpu.VMEM((P_W, D), jnp.float32),
        pltpu.VMEM((B * P_W, D), jnp.float32),
        pltpu.SemaphoreType.DMA((B,)),
        pltpu.SemaphoreType.DMA,
        pltpu.SemaphoreType.DMA((B,)),
        pltpu.SemaphoreType.DMA((B * N_WCH,)),
    ],
)
def _emb_lookup(idx_hbm, tok_hbm, pos_hbm, out_hbm,
                idx_v, pos_v, rows_v, sem_i, sem_p, sem_g, sem_w):
    c = lax.axis_index("c")
    s = lax.axis_index("s")
    wid = s * NC + c
    p0 = wid * P_W

    idx_cps = [
        pltpu.async_copy(
            idx_hbm.at[b, pl.ds(p0, P_W)], idx_v.at[b], sem_i.at[b]
        )
        for b in range(B)
    ]
    pos_cp = pltpu.async_copy(pos_hbm.at[pl.ds(p0, P_W)], pos_v, sem_p)

    gathers = []
    for b in range(B):
        idx_cps[b].wait()
        gathers.append(
            pltpu.async_copy(
                tok_hbm.at[idx_v.at[b]],
                rows_v.at[pl.ds(b * P_W, P_W)],
                sem_g.at[b],
            )
        )
    pos_cp.wait()

    writes = []
    for b in range(B):
        gathers[b].wait()
        for h in range(N_WCH):
            r0 = b * P_W + h * W_CH

            def add_body(j, carry, _r0=r0, _h=h):
                for k in range(D // LANES):
                    sl = pl.ds(k * LANES, LANES)
                    rows_v[_r0 + j, sl] = (
                        rows_v[_r0 + j, sl] + pos_v[_h * W_CH + j, sl]
                    )
                return carry

            lax.fori_loop(0, W_CH, add_body, 0)
            writes.append(
                pltpu.async_copy(
                    rows_v.at[pl.ds(r0, W_CH)],
                    out_hbm.at[pl.ds(b * L_CTX + p0 + h * W_CH, W_CH)],
                    sem_w.at[b * N_WCH + h],
                )
            )
    for w in writes:
        w.wait()


def kernel(inputs, token_table, position_table):
    out = _emb_lookup(inputs.astype(jnp.int32), token_table, position_table)
    return out.reshape(B, L_CTX, D)


# speedup vs baseline: 1.0360x; 1.0360x over previous
import functools

import jax
import jax.numpy as jnp
from jax import lax
from jax.experimental import pallas as pl
from jax.experimental.pallas import tpu as pltpu
from jax.experimental.pallas import tpu_sc as plsc

L_CTX = 2048
D = 128
B = 4
N = B * L_CTX
NC = 2
NS = 16
NW = NC * NS
P_W = L_CTX // NW
W_CH = 32
N_WCH = P_W // W_CH
LANES = 16

_mesh = plsc.VectorSubcoreMesh(core_axis_name="c", subcore_axis_name="s")


@functools.partial(
    pl.kernel,
    out_type=jax.ShapeDtypeStruct((N, D), jnp.float32),
    mesh=_mesh,
    scratch_types=[
        pltpu.VMEM((B, P_W), jnp.int32),
        pltpu.VMEM((P_W, D), jnp.float32),
        pltpu.VMEM((P_W, D), jnp.float32),
        pltpu.VMEM((B * P_W, D), jnp.float32),
        pltpu.SemaphoreType.DMA((B,)),
        pltpu.SemaphoreType.DMA,
        pltpu.SemaphoreType.DMA((B,)),
        pltpu.SemaphoreType.DMA((B * N_WCH,)),
    ],
)
def _emb_lookup(idx_hbm, tok_hbm, pos_hbm, out_hbm,
                idx_v, pos_v, g0_v, rows_v, sem_i, sem_p, sem_g, sem_w):
    c = lax.axis_index("c")
    s = lax.axis_index("s")
    wid = s * NC + c
    p0 = wid * P_W

    idx_cps = [
        pltpu.async_copy(
            idx_hbm.at[b, pl.ds(p0, P_W)], idx_v.at[b], sem_i.at[b]
        )
        for b in range(B)
    ]
    pos_cp = pltpu.async_copy(pos_hbm.at[pl.ds(p0, P_W)], pos_v, sem_p)

    idx_cps[0].wait()
    gathers = [
        pltpu.async_copy(tok_hbm.at[idx_v.at[0]], g0_v, sem_g.at[0])
    ]
    pos_cp.wait()

    for b in range(1, B):
        @plsc.parallel_loop(0, P_W, unroll=4)
        def rep_body(j, _b=b):
            for k in range(D // LANES):
                sl = pl.ds(k * LANES, LANES)
                rows_v[_b * P_W + j, sl] = pos_v[j, sl]

        idx_cps[b].wait()
        gathers.append(
            pltpu.async_copy(
                tok_hbm.at[idx_v.at[b]],
                rows_v.at[pl.ds(b * P_W, P_W)],
                sem_g.at[b],
                add=True,
            )
        )

    writes = []

    def emit_writes(src, b):
        for h in range(N_WCH):
            writes.append(
                pltpu.async_copy(
                    src.at[pl.ds(h * W_CH, W_CH)],
                    out_hbm.at[pl.ds(b * L_CTX + p0 + h * W_CH, W_CH)],
                    sem_w.at[b * N_WCH + h],
                )
            )

    gathers[0].wait()

    @plsc.parallel_loop(0, P_W, unroll=4)
    def add_body(j):
        for k in range(D // LANES):
            sl = pl.ds(k * LANES, LANES)
            g0_v[j, sl] = g0_v[j, sl] + pos_v[j, sl]

    emit_writes(g0_v, 0)

    for b in range(1, B):
        gathers[b].wait()
        emit_writes(rows_v.at[pl.ds(b * P_W, P_W)], b)

    for w in writes:
        w.wait()


def kernel(inputs, token_table, position_table):
    out = _emb_lookup(inputs.astype(jnp.int32), token_table, position_table)
    return out.reshape(B, L_CTX, D)


# speedup vs baseline: 1.0566x; 1.0198x over previous
import functools

import jax
import jax.numpy as jnp
from jax import lax
from jax.experimental import pallas as pl
from jax.experimental.pallas import tpu as pltpu
from jax.experimental.pallas import tpu_sc as plsc

L_CTX = 2048
D = 128
B = 4
N = B * L_CTX
NC = 2
NS = 16
NW = NC * NS
P_W = L_CTX // NW
W_CH = 32
N_WCH = P_W // W_CH
LANES = 16

_POS = 0
_ACC = P_W

_SEM_IDX = 0
_SEM_POS = B
_SEM_G = B + 1
_SEM_W = 2 * B + 1
_N_SEM = 2 * B + 1 + B * N_WCH

_mesh = plsc.VectorSubcoreMesh(core_axis_name="c", subcore_axis_name="s")


@functools.partial(
    pl.kernel,
    out_type=jax.ShapeDtypeStruct((N, D), jnp.float32),
    mesh=_mesh,
    scratch_types=[
        pltpu.VMEM((B, P_W), jnp.int32),
        pltpu.VMEM(((B + 1) * P_W, D), jnp.float32),
        pltpu.SemaphoreType.DMA((_N_SEM,)),
    ],
)
def _emb_lookup(idx_hbm, tok_hbm, pos_hbm, out_hbm, idx_v, fbuf, sem):
    c = lax.axis_index("c")
    s = lax.axis_index("s")
    wid = s * NC + c
    p0 = wid * P_W

    idx_cps = [
        pltpu.async_copy(
            idx_hbm.at[b, pl.ds(p0, P_W)], idx_v.at[b], sem.at[_SEM_IDX + b]
        )
        for b in range(B)
    ]
    pos_cp = pltpu.async_copy(
        pos_hbm.at[pl.ds(p0, P_W)], fbuf.at[pl.ds(_POS, P_W)], sem.at[_SEM_POS]
    )

    idx_cps[0].wait()
    gathers = [
        pltpu.async_copy(
            tok_hbm.at[idx_v.at[0]], fbuf.at[pl.ds(_ACC, P_W)], sem.at[_SEM_G]
        )
    ]
    pos_cp.wait()

    for b in range(1, B):
        def rep_body(j, carry, _b=b):
            for k in range(D // LANES):
                sl = pl.ds(k * LANES, LANES)
                fbuf[_ACC + _b * P_W + j, sl] = fbuf[_POS + j, sl]
            return carry

        lax.fori_loop(0, P_W, rep_body, 0)
        idx_cps[b].wait()
        gathers.append(
            pltpu.async_copy(
                tok_hbm.at[idx_v.at[b]],
                fbuf.at[pl.ds(_ACC + b * P_W, P_W)],
                sem.at[_SEM_G + b],
                add=True,
            )
        )

    writes = []

    def emit_writes(b):
        for h in range(N_WCH):
            writes.append(
                pltpu.async_copy(
                    fbuf.at[pl.ds(_ACC + b * P_W + h * W_CH, W_CH)],
                    out_hbm.at[pl.ds(b * L_CTX + p0 + h * W_CH, W_CH)],
                    sem.at[_SEM_W + b * N_WCH + h],
                )
            )

    gathers[0].wait()

    def add_body(j, carry):
        for k in range(D // LANES):
            sl = pl.ds(k * LANES, LANES)
            fbuf[_ACC + j, sl] = fbuf[_ACC + j, sl] + fbuf[_POS + j, sl]
        return carry

    lax.fori_loop(0, P_W, add_body, 0)
    emit_writes(0)

    for b in range(1, B):
        gathers[b].wait()
        emit_writes(b)

    for w in writes:
        w.wait()


def kernel(inputs, token_table, position_table):
    out = _emb_lookup(inputs.astype(jnp.int32), token_table, position_table)
    return out.reshape(B, L_CTX, D)
